# baseline (speedup 1.0000x reference)
import jax
import jax.numpy as jnp
from jax import lax
from jax.experimental import pallas as pl
from jax.experimental.pallas import tpu as pltpu
from jax.experimental.pallas import tpu_sc as plsc

N = 10000
N2 = 10240
E = 320000
F_IN = 128
H = 128
C = 64

NC = 2
NS = 16
NW = NC * NS
EPT = E // NW
CH = 50
CPW = EPT // CH
PH = 40
NPH = CPW // PH
STR = N2 // NS
ZR = 8

BLK = 512
GRID = N2 // BLK

_MESH = dict(core_axis_name="c", subcore_axis_name="s", num_cores=NC,
             num_subcores=NS)


def _sc_deg_body(src_hbm, dst_hbm, out_hbm, sbuf, dbuf, cin, cout):
    w = lax.axis_index("s") * NC + lax.axis_index("c")
    base = w * EPT
    z = jnp.zeros((16,), jnp.float32)

    @pl.loop(0, N2 // 16, unroll=4)
    def _zero(i):
        cin[0, pl.ds(i * 16, 16)] = z
        cout[0, pl.ds(i * 16, 16)] = z

    pltpu.sync_copy(src_hbm.at[pl.ds(base, EPT)], sbuf)
    pltpu.sync_copy(dst_hbm.at[pl.ds(base, EPT)], dbuf)

    ones = jnp.ones((16,), jnp.float32)
    zi = jnp.zeros((16,), jnp.int32)

    @plsc.parallel_loop(0, EPT // 16, unroll=4)
    def _count(i):
        s16 = sbuf[pl.ds(i * 16, 16)]
        d16 = dbuf[pl.ds(i * 16, 16)]
        plsc.addupdate_scatter(cout, [zi, s16], ones)
        plsc.addupdate_scatter(cin, [zi, d16], ones)

    pltpu.sync_copy(cin.at[0], out_hbm.at[w, 0])
    pltpu.sync_copy(cout.at[0], out_hbm.at[w, 1])


def _sc_deg(src, dst):
    return pl.kernel(
        _sc_deg_body,
        out_type=jax.ShapeDtypeStruct((NW, 2, N2), jnp.float32),
        mesh=plsc.VectorSubcoreMesh(**_MESH),
        compiler_params=pltpu.CompilerParams(needs_layout_passes=False),
        scratch_types=[
            pltpu.VMEM((EPT,), jnp.int32),
            pltpu.VMEM((EPT,), jnp.int32),
            pltpu.VMEM((1, N2), jnp.float32),
            pltpu.VMEM((1, N2), jnp.float32),
        ],
    )(src, dst)


def _sc_agg_body(x_hbm, src2_hbm, dst2_hbm, out_hbm,
                 sbufA, dbufA, sbufB, dbufB, rows0, rows1, zbuf, acc_sp,
                 semg0, semg1, semi):
    c = lax.axis_index("c")
    s = lax.axis_index("s")
    tile = c * NS + s

    z = jnp.zeros((16,), jnp.float32)

    @pl.loop(0, ZR)
    def _zr(r):
        for l in range(H // 16):
            zbuf[r, pl.ds(l * 16, 16)] = z

    row0 = s * STR

    @pl.loop(0, STR // ZR)
    def _zs(i):
        pltpu.sync_copy(zbuf, acc_sp.at[pl.ds(row0 + i * ZR, ZR)])

    plsc.subcore_barrier()

    rows = (rows0, rows1)
    sems = (semg0, semg1)
    sbufs = (sbufA, sbufB)
    dbufs = (dbufA, dbufB)

    def _gather(sbuf, k, b):
        pltpu.make_async_copy(x_hbm.at[sbuf.at[k]], rows[b], sems[b]).start()

    def _gwait(b):
        pltpu.make_async_copy(x_hbm.at[sbufA.at[0]], rows[b], sems[b]).wait()

    def _scatter(dbuf, k, b):
        pltpu.sync_copy(rows[b], acc_sp.at[dbuf.at[k]], add=True)

    def _idx_load(p, q):
        base = tile * CPW + p * PH
        pltpu.make_async_copy(
            src2_hbm.at[pl.ds(base, PH)], sbufs[q], semi).start()
        pltpu.make_async_copy(
            dst2_hbm.at[pl.ds(base, PH)], dbufs[q], semi).start()

    def _idx_wait(q):
        pltpu.make_async_copy(
            src2_hbm.at[pl.ds(0, PH)], sbufs[q], semi).wait()
        pltpu.make_async_copy(
            src2_hbm.at[pl.ds(0, PH)], dbufs[q], semi).wait()

    _idx_load(0, 0)

    for p in range(NPH):
        q = p % 2
        sbuf = sbufs[q]
        dbuf = dbufs[q]
        _idx_wait(q)
        _gather(sbuf, 0, 0)
        _gather(sbuf, 1, 1)
        if p + 1 < NPH:
            _idx_load(p + 1, 1 - q)

        @pl.loop(0, PH // 2 - 1)
        def _main(g):
            k0 = g * 2
            _gwait(0)
            _scatter(dbuf, k0, 0)
            _gather(sbuf, k0 + 2, 0)
            _gwait(1)
            _scatter(dbuf, k0 + 1, 1)
            _gather(sbuf, k0 + 3, 1)

        _gwait(0)
        _scatter(dbuf, PH - 2, 0)
        _gwait(1)
        _scatter(dbuf, PH - 1, 1)

    plsc.subcore_barrier()
    pltpu.sync_copy(acc_sp.at[pl.ds(row0, STR)],
                    out_hbm.at[c, pl.ds(row0, STR)])


def _sc_agg(x, src2, dst2):
    return pl.kernel(
        _sc_agg_body,
        out_type=jax.ShapeDtypeStruct((NC, N2, H), jnp.float32),
        mesh=plsc.VectorSubcoreMesh(**_MESH),
        compiler_params=pltpu.CompilerParams(needs_layout_passes=False),
        scratch_types=[
            pltpu.VMEM((PH, CH), jnp.int32),
            pltpu.VMEM((PH, CH), jnp.int32),
            pltpu.VMEM((PH, CH), jnp.int32),
            pltpu.VMEM((PH, CH), jnp.int32),
            pltpu.VMEM((CH, H), jnp.float32),
            pltpu.VMEM((CH, H), jnp.float32),
            pltpu.VMEM((ZR, H), jnp.float32),
            pltpu.VMEM_SHARED((N2, H), jnp.float32),
            pltpu.SemaphoreType.DMA,
            pltpu.SemaphoreType.DMA,
            pltpu.SemaphoreType.DMA,
        ],
    )(x, src2, dst2)


def _tc_in_body(x_ref, w_ref, b_ref, h_ref):
    h = jnp.dot(x_ref[...], w_ref[...], preferred_element_type=jnp.float32)
    h_ref[...] = jnp.maximum(h + b_ref[...], 0.0)


def _tc_in(x, W_in, b_in):
    return pl.pallas_call(
        _tc_in_body,
        grid=(GRID,),
        in_specs=[
            pl.BlockSpec((BLK, F_IN), lambda i: (i, 0)),
            pl.BlockSpec((F_IN, H), lambda i: (0, 0)),
            pl.BlockSpec((1, H), lambda i: (0, 0)),
        ],
        out_specs=pl.BlockSpec((BLK, H), lambda i: (i, 0)),
        out_shape=jax.ShapeDtypeStruct((N2, H), jnp.float32),
    )(x, W_in, b_in.reshape(1, H))


def _deg_from_cnt(cnt_ref):
    cnt = jnp.sum(cnt_ref[...], axis=0)
    deg_in = jnp.maximum(cnt[0:1], 1.0)
    deg_out = jnp.maximum(cnt[1:2], 1.0)
    return deg_in, deg_out


def _tc_mid_body(cnt_ref, h_ref, nmp_ref, wn_ref, bn_ref, wa1_ref, ba1_ref,
                 wa2_ref, ba2_ref, xs_ref):
    deg_in, deg_out = _deg_from_cnt(cnt_ref)
    inv_in = (1.0 / deg_in).T
    dso = lax.rsqrt(deg_out).T
    h = h_ref[...]
    nm = (nmp_ref[0] + nmp_ref[1]) * inv_in
    agg = (jnp.dot(h, wn_ref[:H], preferred_element_type=jnp.float32)
           + jnp.dot(nm, wn_ref[H:], preferred_element_type=jnp.float32)
           + bn_ref[...])
    agg = jnp.maximum(agg, 0.0)
    z = (jnp.dot(h, wa1_ref[:H], preferred_element_type=jnp.float32)
         + jnp.dot(agg, wa1_ref[H:], preferred_element_type=jnp.float32)
         + ba1_ref[...])
    z = jnp.maximum(z, 0.0)
    logit = jnp.sum(z * wa2_ref[...], axis=1, keepdims=True) + ba2_ref[...]
    att = jax.nn.sigmoid(logit)
    xs_ref[...] = (h + agg * att) * dso


def _tc_mid(cnts, h, nmp, Wn, bn_b, Wa1, ba1, Wa2, ba2):
    return pl.pallas_call(
        _tc_mid_body,
        grid=(GRID,),
        in_specs=[
            pl.BlockSpec((NW, 2, BLK), lambda i: (0, 0, i)),
            pl.BlockSpec((BLK, H), lambda i: (i, 0)),
            pl.BlockSpec((NC, BLK, H), lambda i: (0, i, 0)),
            pl.BlockSpec((2 * H, H), lambda i: (0, 0)),
            pl.BlockSpec((1, H), lambda i: (0, 0)),
            pl.BlockSpec((2 * H, H), lambda i: (0, 0)),
            pl.BlockSpec((1, H), lambda i: (0, 0)),
            pl.BlockSpec((1, H), lambda i: (0, 0)),
            pl.BlockSpec((1, 1), lambda i: (0, 0)),
        ],
        out_specs=pl.BlockSpec((BLK, H), lambda i: (i, 0)),
        out_shape=jax.ShapeDtypeStruct((N2, H), jnp.float32),
    )(cnts, h, nmp, Wn, bn_b.reshape(1, H), Wa1, ba1.reshape(1, H),
      Wa2.reshape(1, H), ba2.reshape(1, 1))


def _tc_conv_body(cnt_ref, sp_ref, wg_ref, bg_ref, t_ref, st_ref):
    i = pl.program_id(0)
    deg_in, _ = _deg_from_cnt(cnt_ref)
    dsi = lax.rsqrt(deg_in).T
    sblk = (sp_ref[0] + sp_ref[1]) * dsi
    t = jnp.dot(sblk, wg_ref[...], preferred_element_type=jnp.float32)
    t = t + bg_ref[...]
    t_ref[...] = t

    @pl.when(i == 0)
    def _():
        st_ref[...] = jnp.zeros((8, H), jnp.float32)
        st_ref[2:3, :] = jnp.sum(t, axis=0, keepdims=True) * (1.0 / BLK)

    shift = st_ref[2:3, :]
    rows = lax.broadcasted_iota(jnp.int32, (BLK, 1), 0) + i * BLK
    tm = jnp.where(rows < N, t - shift, 0.0)
    st_ref[0:1, :] += jnp.sum(tm, axis=0, keepdims=True)
    st_ref[1:2, :] += jnp.sum(tm * tm, axis=0, keepdims=True)


def _tc_conv(cnts, sp, Wg, bg):
    return pl.pallas_call(
        _tc_conv_body,
        grid=(GRID,),
        in_specs=[
            pl.BlockSpec((NW, 2, BLK), lambda i: (0, 0, i)),
            pl.BlockSpec((NC, BLK, H), lambda i: (0, i, 0)),
            pl.BlockSpec((H, H), lambda i: (0, 0)),
            pl.BlockSpec((1, H), lambda i: (0, 0)),
        ],
        out_specs=[
            pl.BlockSpec((BLK, H), lambda i: (i, 0)),
            pl.BlockSpec((8, H), lambda i: (0, 0)),
        ],
        out_shape=[
            jax.ShapeDtypeStruct((N2, H), jnp.float32),
            jax.ShapeDtypeStruct((8, H), jnp.float32),
        ],
    )(cnts, sp, Wg, bg.reshape(1, H))


def _bn_relu(t_ref, st_ref, g_ref, be_ref):
    d = st_ref[0:1, :] * (1.0 / N)
    ex2 = st_ref[1:2, :] * (1.0 / N)
    mu = st_ref[2:3, :] + d
    var = ex2 - d * d
    scale = g_ref[...] * lax.rsqrt(var + 1e-5)
    return jnp.maximum((t_ref[...] - mu) * scale + be_ref[...], 0.0)


def _tc_bn_body(cnt_ref, t_ref, st_ref, g_ref, be_ref, h_ref, xs_ref):
    _, deg_out = _deg_from_cnt(cnt_ref)
    dso = lax.rsqrt(deg_out).T
    hn = _bn_relu(t_ref, st_ref, g_ref, be_ref)
    h_ref[...] = hn
    xs_ref[...] = hn * dso


def _tc_bn_res_body(cnt_ref, t_ref, st_ref, g_ref, be_ref, hp_ref,
                    h_ref, xs_ref):
    _, deg_out = _deg_from_cnt(cnt_ref)
    dso = lax.rsqrt(deg_out).T
    hn = _bn_relu(t_ref, st_ref, g_ref, be_ref) + hp_ref[...]
    h_ref[...] = hn
    xs_ref[...] = hn * dso


_BN_IN_SPECS = [
    pl.BlockSpec((NW, 2, BLK), lambda i: (0, 0, i)),
    pl.BlockSpec((BLK, H), lambda i: (i, 0)),
    pl.BlockSpec((8, H), lambda i: (0, 0)),
    pl.BlockSpec((1, H), lambda i: (0, 0)),
    pl.BlockSpec((1, H), lambda i: (0, 0)),
]
_BN_OUT_SPECS = [
    pl.BlockSpec((BLK, H), lambda i: (i, 0)),
    pl.BlockSpec((BLK, H), lambda i: (i, 0)),
]
_BN_OUT_SHAPE = [
    jax.ShapeDtypeStruct((N2, H), jnp.float32),
    jax.ShapeDtypeStruct((N2, H), jnp.float32),
]


def _tc_bn(cnts, t, st, g, be):
    return pl.pallas_call(
        _tc_bn_body, grid=(GRID,), in_specs=_BN_IN_SPECS,
        out_specs=_BN_OUT_SPECS, out_shape=_BN_OUT_SHAPE,
    )(cnts, t, st, g.reshape(1, H), be.reshape(1, H))


def _tc_bn_res(cnts, t, st, g, be, hprev):
    return pl.pallas_call(
        _tc_bn_res_body, grid=(GRID,),
        in_specs=_BN_IN_SPECS + [pl.BlockSpec((BLK, H), lambda i: (i, 0))],
        out_specs=_BN_OUT_SPECS, out_shape=_BN_OUT_SHAPE,
    )(cnts, t, st, g.reshape(1, H), be.reshape(1, H), hprev)


def _tc_bn_final_body(t_ref, st_ref, g_ref, be_ref, hp_ref, wo_ref, bo_ref,
                      out_ref):
    hn = _bn_relu(t_ref, st_ref, g_ref, be_ref) + hp_ref[...]
    out_ref[...] = (jnp.dot(hn, wo_ref[...], preferred_element_type=jnp.float32)
                    + bo_ref[...])


def _tc_bn_final(t, st, g, be, hprev, Wo, bo):
    return pl.pallas_call(
        _tc_bn_final_body,
        grid=(GRID,),
        in_specs=[
            pl.BlockSpec((BLK, H), lambda i: (i, 0)),
            pl.BlockSpec((8, H), lambda i: (0, 0)),
            pl.BlockSpec((1, H), lambda i: (0, 0)),
            pl.BlockSpec((1, H), lambda i: (0, 0)),
            pl.BlockSpec((BLK, H), lambda i: (i, 0)),
            pl.BlockSpec((H, C), lambda i: (0, 0)),
            pl.BlockSpec((1, C), lambda i: (0, 0)),
        ],
        out_specs=pl.BlockSpec((BLK, C), lambda i: (i, 0)),
        out_shape=jax.ShapeDtypeStruct((N2, C), jnp.float32),
    )(t, st, g.reshape(1, H), be.reshape(1, H), hprev, Wo, bo.reshape(1, C))


def kernel(features, edge_index, W_in, b_in, Wn, bn_b, Wa1, ba1, Wa2, ba2,
           Wg0, bg0, Wg1, bg1, Wg2, bg2, g0, be0, g1, be1, g2, be2, Wo, bo):
    src = edge_index[0]
    dst = edge_index[1]
    src2 = src.reshape(E // CH, CH)
    dst2 = dst.reshape(E // CH, CH)
    xpad = jnp.pad(features, ((0, N2 - N), (0, 0)))

    cnts = _sc_deg(src, dst)
    h = _tc_in(xpad, W_in, b_in)
    nmp = _sc_agg(h, src2, dst2)
    xs = _tc_mid(cnts, h, nmp, Wn, bn_b, Wa1, ba1, Wa2, ba2)

    sp = _sc_agg(xs, src2, dst2)
    t0, st0 = _tc_conv(cnts, sp, Wg0, bg0)
    h1, xs = _tc_bn(cnts, t0, st0, g0, be0)

    sp = _sc_agg(xs, src2, dst2)
    t1, st1 = _tc_conv(cnts, sp, Wg1, bg1)
    h2, xs = _tc_bn_res(cnts, t1, st1, g1, be1, h1)

    sp = _sc_agg(xs, src2, dst2)
    t2, st2 = _tc_conv(cnts, sp, Wg2, bg2)
    out = _tc_bn_final(t2, st2, g2, be2, h2, Wo, bo)
    return out[:N]


# speedup vs baseline: 1.0213x; 1.0213x over previous
import jax
import jax.numpy as jnp
from jax import lax
from jax.experimental import pallas as pl
from jax.experimental.pallas import tpu as pltpu
from jax.experimental.pallas import tpu_sc as plsc

N = 10000
N2 = 10240
E = 320000
F_IN = 128
H = 128
C = 64

NC = 2
NS = 16
NW = NC * NS
EPT = E // NW
CH = 50
CPW = EPT // CH
PH = 40
NPH = CPW // PH
STR = N2 // NS
ZR = 8

BLK = 512
GRID = N2 // BLK

_MESH = dict(core_axis_name="c", subcore_axis_name="s", num_cores=NC,
             num_subcores=NS)


def _sc_deg_body(src_hbm, dst_hbm, out_hbm, sbuf, dbuf, cin, cout):
    w = lax.axis_index("s") * NC + lax.axis_index("c")
    base = w * EPT
    z = jnp.zeros((16,), jnp.float32)

    @pl.loop(0, N2 // 16, unroll=4)
    def _zero(i):
        cin[0, pl.ds(i * 16, 16)] = z
        cout[0, pl.ds(i * 16, 16)] = z

    pltpu.sync_copy(src_hbm.at[pl.ds(base, EPT)], sbuf)
    pltpu.sync_copy(dst_hbm.at[pl.ds(base, EPT)], dbuf)

    ones = jnp.ones((16,), jnp.float32)
    zi = jnp.zeros((16,), jnp.int32)

    @plsc.parallel_loop(0, EPT // 16, unroll=4)
    def _count(i):
        s16 = sbuf[pl.ds(i * 16, 16)]
        d16 = dbuf[pl.ds(i * 16, 16)]
        plsc.addupdate_scatter(cout, [zi, s16], ones)
        plsc.addupdate_scatter(cin, [zi, d16], ones)

    pltpu.sync_copy(cin.at[0], out_hbm.at[w, 0])
    pltpu.sync_copy(cout.at[0], out_hbm.at[w, 1])


def _sc_deg(src, dst):
    return pl.kernel(
        _sc_deg_body,
        out_type=jax.ShapeDtypeStruct((NW, 2, N2), jnp.float32),
        mesh=plsc.VectorSubcoreMesh(**_MESH),
        compiler_params=pltpu.CompilerParams(needs_layout_passes=False),
        scratch_types=[
            pltpu.VMEM((EPT,), jnp.int32),
            pltpu.VMEM((EPT,), jnp.int32),
            pltpu.VMEM((1, N2), jnp.float32),
            pltpu.VMEM((1, N2), jnp.float32),
        ],
    )(src, dst)


def _sc_agg_body(x_hbm, src2_hbm, dst2_hbm, out_hbm,
                 sbufA, dbufA, sbufB, dbufB, rows0, rows1, zbuf, acc_sp,
                 semg0, semg1, semi):
    c = lax.axis_index("c")
    s = lax.axis_index("s")
    tile = c * NS + s

    z = jnp.zeros((16,), jnp.float32)

    @pl.loop(0, ZR)
    def _zr(r):
        for l in range(H // 16):
            zbuf[r, pl.ds(l * 16, 16)] = z

    row0 = s * STR

    @pl.loop(0, STR // ZR)
    def _zs(i):
        pltpu.make_async_copy(
            zbuf, acc_sp.at[pl.ds(row0 + i * ZR, ZR)], semi).start()

    @pl.loop(0, STR // ZR)
    def _zw(i):
        pltpu.make_async_copy(
            zbuf, acc_sp.at[pl.ds(row0, ZR)], semi).wait()

    plsc.subcore_barrier()

    rows = (rows0, rows1)
    sems = (semg0, semg1)
    sbufs = (sbufA, sbufB)
    dbufs = (dbufA, dbufB)

    def _gather(sbuf, k, b):
        pltpu.make_async_copy(x_hbm.at[sbuf.at[k]], rows[b], sems[b]).start()

    def _gwait(b):
        pltpu.make_async_copy(x_hbm.at[sbufA.at[0]], rows[b], sems[b]).wait()

    def _scatter(dbuf, k, b):
        pltpu.sync_copy(rows[b], acc_sp.at[dbuf.at[k]], add=True)

    def _idx_load(p, q):
        base = tile * CPW + p * PH
        pltpu.make_async_copy(
            src2_hbm.at[pl.ds(base, PH)], sbufs[q], semi).start()
        pltpu.make_async_copy(
            dst2_hbm.at[pl.ds(base, PH)], dbufs[q], semi).start()

    def _idx_wait(q):
        pltpu.make_async_copy(
            src2_hbm.at[pl.ds(0, PH)], sbufs[q], semi).wait()
        pltpu.make_async_copy(
            src2_hbm.at[pl.ds(0, PH)], dbufs[q], semi).wait()

    _idx_load(0, 0)

    for p in range(NPH):
        q = p % 2
        sbuf = sbufs[q]
        dbuf = dbufs[q]
        _idx_wait(q)
        _gather(sbuf, 0, 0)
        _gather(sbuf, 1, 1)
        if p + 1 < NPH:
            _idx_load(p + 1, 1 - q)

        @pl.loop(0, PH // 2 - 1)
        def _main(g):
            k0 = g * 2
            _gwait(0)
            _scatter(dbuf, k0, 0)
            _gather(sbuf, k0 + 2, 0)
            _gwait(1)
            _scatter(dbuf, k0 + 1, 1)
            _gather(sbuf, k0 + 3, 1)

        _gwait(0)
        _scatter(dbuf, PH - 2, 0)
        _gwait(1)
        _scatter(dbuf, PH - 1, 1)

    plsc.subcore_barrier()
    pltpu.sync_copy(acc_sp.at[pl.ds(row0, STR)],
                    out_hbm.at[c, pl.ds(row0, STR)])


def _sc_agg(x, src2, dst2):
    return pl.kernel(
        _sc_agg_body,
        out_type=jax.ShapeDtypeStruct((NC, N2, H), jnp.float32),
        mesh=plsc.VectorSubcoreMesh(**_MESH),
        compiler_params=pltpu.CompilerParams(needs_layout_passes=False),
        scratch_types=[
            pltpu.VMEM((PH, CH), jnp.int32),
            pltpu.VMEM((PH, CH), jnp.int32),
            pltpu.VMEM((PH, CH), jnp.int32),
            pltpu.VMEM((PH, CH), jnp.int32),
            pltpu.VMEM((CH, H), jnp.float32),
            pltpu.VMEM((CH, H), jnp.float32),
            pltpu.VMEM((ZR, H), jnp.float32),
            pltpu.VMEM_SHARED((N2, H), jnp.float32),
            pltpu.SemaphoreType.DMA,
            pltpu.SemaphoreType.DMA,
            pltpu.SemaphoreType.DMA,
        ],
    )(x, src2, dst2)


def _tc_in_body(x_ref, w_ref, b_ref, h_ref):
    h = jnp.dot(x_ref[...], w_ref[...], preferred_element_type=jnp.float32)
    h_ref[...] = jnp.maximum(h + b_ref[...], 0.0)


def _tc_in(x, W_in, b_in):
    return pl.pallas_call(
        _tc_in_body,
        grid=(GRID,),
        in_specs=[
            pl.BlockSpec((BLK, F_IN), lambda i: (i, 0)),
            pl.BlockSpec((F_IN, H), lambda i: (0, 0)),
            pl.BlockSpec((1, H), lambda i: (0, 0)),
        ],
        out_specs=pl.BlockSpec((BLK, H), lambda i: (i, 0)),
        out_shape=jax.ShapeDtypeStruct((N2, H), jnp.float32),
    )(x, W_in, b_in.reshape(1, H))


def _deg_from_cnt(cnt_ref):
    cnt = jnp.sum(cnt_ref[...], axis=0)
    deg_in = jnp.maximum(cnt[0:1], 1.0)
    deg_out = jnp.maximum(cnt[1:2], 1.0)
    return deg_in, deg_out


def _tc_mid_body(cnt_ref, h_ref, nmp_ref, wn_ref, bn_ref, wa1_ref, ba1_ref,
                 wa2_ref, ba2_ref, xs_ref):
    deg_in, deg_out = _deg_from_cnt(cnt_ref)
    inv_in = (1.0 / deg_in).T
    dso = lax.rsqrt(deg_out).T
    h = h_ref[...]
    nm = (nmp_ref[0] + nmp_ref[1]) * inv_in
    agg = (jnp.dot(h, wn_ref[:H], preferred_element_type=jnp.float32)
           + jnp.dot(nm, wn_ref[H:], preferred_element_type=jnp.float32)
           + bn_ref[...])
    agg = jnp.maximum(agg, 0.0)
    z = (jnp.dot(h, wa1_ref[:H], preferred_element_type=jnp.float32)
         + jnp.dot(agg, wa1_ref[H:], preferred_element_type=jnp.float32)
         + ba1_ref[...])
    z = jnp.maximum(z, 0.0)
    logit = jnp.sum(z * wa2_ref[...], axis=1, keepdims=True) + ba2_ref[...]
    att = jax.nn.sigmoid(logit)
    xs_ref[...] = (h + agg * att) * dso


def _tc_mid(cnts, h, nmp, Wn, bn_b, Wa1, ba1, Wa2, ba2):
    return pl.pallas_call(
        _tc_mid_body,
        grid=(GRID,),
        in_specs=[
            pl.BlockSpec((NW, 2, BLK), lambda i: (0, 0, i)),
            pl.BlockSpec((BLK, H), lambda i: (i, 0)),
            pl.BlockSpec((NC, BLK, H), lambda i: (0, i, 0)),
            pl.BlockSpec((2 * H, H), lambda i: (0, 0)),
            pl.BlockSpec((1, H), lambda i: (0, 0)),
            pl.BlockSpec((2 * H, H), lambda i: (0, 0)),
            pl.BlockSpec((1, H), lambda i: (0, 0)),
            pl.BlockSpec((1, H), lambda i: (0, 0)),
            pl.BlockSpec((1, 1), lambda i: (0, 0)),
        ],
        out_specs=pl.BlockSpec((BLK, H), lambda i: (i, 0)),
        out_shape=jax.ShapeDtypeStruct((N2, H), jnp.float32),
    )(cnts, h, nmp, Wn, bn_b.reshape(1, H), Wa1, ba1.reshape(1, H),
      Wa2.reshape(1, H), ba2.reshape(1, 1))


def _tc_conv_body(cnt_ref, sp_ref, wg_ref, bg_ref, t_ref, st_ref):
    i = pl.program_id(0)
    deg_in, _ = _deg_from_cnt(cnt_ref)
    dsi = lax.rsqrt(deg_in).T
    sblk = (sp_ref[0] + sp_ref[1]) * dsi
    t = jnp.dot(sblk, wg_ref[...], preferred_element_type=jnp.float32)
    t = t + bg_ref[...]
    t_ref[...] = t

    @pl.when(i == 0)
    def _():
        st_ref[...] = jnp.zeros((8, H), jnp.float32)
        st_ref[2:3, :] = jnp.sum(t, axis=0, keepdims=True) * (1.0 / BLK)

    shift = st_ref[2:3, :]
    rows = lax.broadcasted_iota(jnp.int32, (BLK, 1), 0) + i * BLK
    tm = jnp.where(rows < N, t - shift, 0.0)
    st_ref[0:1, :] += jnp.sum(tm, axis=0, keepdims=True)
    st_ref[1:2, :] += jnp.sum(tm * tm, axis=0, keepdims=True)


def _tc_conv(cnts, sp, Wg, bg):
    return pl.pallas_call(
        _tc_conv_body,
        grid=(GRID,),
        in_specs=[
            pl.BlockSpec((NW, 2, BLK), lambda i: (0, 0, i)),
            pl.BlockSpec((NC, BLK, H), lambda i: (0, i, 0)),
            pl.BlockSpec((H, H), lambda i: (0, 0)),
            pl.BlockSpec((1, H), lambda i: (0, 0)),
        ],
        out_specs=[
            pl.BlockSpec((BLK, H), lambda i: (i, 0)),
            pl.BlockSpec((8, H), lambda i: (0, 0)),
        ],
        out_shape=[
            jax.ShapeDtypeStruct((N2, H), jnp.float32),
            jax.ShapeDtypeStruct((8, H), jnp.float32),
        ],
    )(cnts, sp, Wg, bg.reshape(1, H))


def _bn_relu(t_ref, st_ref, g_ref, be_ref):
    d = st_ref[0:1, :] * (1.0 / N)
    ex2 = st_ref[1:2, :] * (1.0 / N)
    mu = st_ref[2:3, :] + d
    var = ex2 - d * d
    scale = g_ref[...] * lax.rsqrt(var + 1e-5)
    return jnp.maximum((t_ref[...] - mu) * scale + be_ref[...], 0.0)


def _tc_bn_body(cnt_ref, t_ref, st_ref, g_ref, be_ref, h_ref, xs_ref):
    _, deg_out = _deg_from_cnt(cnt_ref)
    dso = lax.rsqrt(deg_out).T
    hn = _bn_relu(t_ref, st_ref, g_ref, be_ref)
    h_ref[...] = hn
    xs_ref[...] = hn * dso


def _tc_bn_res_body(cnt_ref, t_ref, st_ref, g_ref, be_ref, hp_ref,
                    h_ref, xs_ref):
    _, deg_out = _deg_from_cnt(cnt_ref)
    dso = lax.rsqrt(deg_out).T
    hn = _bn_relu(t_ref, st_ref, g_ref, be_ref) + hp_ref[...]
    h_ref[...] = hn
    xs_ref[...] = hn * dso


_BN_IN_SPECS = [
    pl.BlockSpec((NW, 2, BLK), lambda i: (0, 0, i)),
    pl.BlockSpec((BLK, H), lambda i: (i, 0)),
    pl.BlockSpec((8, H), lambda i: (0, 0)),
    pl.BlockSpec((1, H), lambda i: (0, 0)),
    pl.BlockSpec((1, H), lambda i: (0, 0)),
]
_BN_OUT_SPECS = [
    pl.BlockSpec((BLK, H), lambda i: (i, 0)),
    pl.BlockSpec((BLK, H), lambda i: (i, 0)),
]
_BN_OUT_SHAPE = [
    jax.ShapeDtypeStruct((N2, H), jnp.float32),
    jax.ShapeDtypeStruct((N2, H), jnp.float32),
]


def _tc_bn(cnts, t, st, g, be):
    return pl.pallas_call(
        _tc_bn_body, grid=(GRID,), in_specs=_BN_IN_SPECS,
        out_specs=_BN_OUT_SPECS, out_shape=_BN_OUT_SHAPE,
    )(cnts, t, st, g.reshape(1, H), be.reshape(1, H))


def _tc_bn_res(cnts, t, st, g, be, hprev):
    return pl.pallas_call(
        _tc_bn_res_body, grid=(GRID,),
        in_specs=_BN_IN_SPECS + [pl.BlockSpec((BLK, H), lambda i: (i, 0))],
        out_specs=_BN_OUT_SPECS, out_shape=_BN_OUT_SHAPE,
    )(cnts, t, st, g.reshape(1, H), be.reshape(1, H), hprev)


def _tc_bn_final_body(t_ref, st_ref, g_ref, be_ref, hp_ref, wo_ref, bo_ref,
                      out_ref):
    hn = _bn_relu(t_ref, st_ref, g_ref, be_ref) + hp_ref[...]
    out_ref[...] = (jnp.dot(hn, wo_ref[...], preferred_element_type=jnp.float32)
                    + bo_ref[...])


def _tc_bn_final(t, st, g, be, hprev, Wo, bo):
    return pl.pallas_call(
        _tc_bn_final_body,
        grid=(GRID,),
        in_specs=[
            pl.BlockSpec((BLK, H), lambda i: (i, 0)),
            pl.BlockSpec((8, H), lambda i: (0, 0)),
            pl.BlockSpec((1, H), lambda i: (0, 0)),
            pl.BlockSpec((1, H), lambda i: (0, 0)),
            pl.BlockSpec((BLK, H), lambda i: (i, 0)),
            pl.BlockSpec((H, C), lambda i: (0, 0)),
            pl.BlockSpec((1, C), lambda i: (0, 0)),
        ],
        out_specs=pl.BlockSpec((BLK, C), lambda i: (i, 0)),
        out_shape=jax.ShapeDtypeStruct((N2, C), jnp.float32),
    )(t, st, g.reshape(1, H), be.reshape(1, H), hprev, Wo, bo.reshape(1, C))


def kernel(features, edge_index, W_in, b_in, Wn, bn_b, Wa1, ba1, Wa2, ba2,
           Wg0, bg0, Wg1, bg1, Wg2, bg2, g0, be0, g1, be1, g2, be2, Wo, bo):
    src = edge_index[0]
    dst = edge_index[1]
    src2 = src.reshape(E // CH, CH)
    dst2 = dst.reshape(E // CH, CH)
    xpad = jnp.pad(features, ((0, N2 - N), (0, 0)))

    cnts = _sc_deg(src, dst)
    h = _tc_in(xpad, W_in, b_in)
    nmp = _sc_agg(h, src2, dst2)
    xs = _tc_mid(cnts, h, nmp, Wn, bn_b, Wa1, ba1, Wa2, ba2)

    sp = _sc_agg(xs, src2, dst2)
    t0, st0 = _tc_conv(cnts, sp, Wg0, bg0)
    h1, xs = _tc_bn(cnts, t0, st0, g0, be0)

    sp = _sc_agg(xs, src2, dst2)
    t1, st1 = _tc_conv(cnts, sp, Wg1, bg1)
    h2, xs = _tc_bn_res(cnts, t1, st1, g1, be1, h1)

    sp = _sc_agg(xs, src2, dst2)
    t2, st2 = _tc_conv(cnts, sp, Wg2, bg2)
    out = _tc_bn_final(t2, st2, g2, be2, h2, Wo, bo)
    return out[:N]
